# baseline (speedup 1.0000x reference)
from functools import partial

import jax
import jax.numpy as jnp
from jax.experimental import pallas as pl
from jax.experimental.pallas import tpu as pltpu


def _fold_bn(gamma, beta, mean, var, eps=1e-5):
    scale = gamma / jnp.sqrt(var + eps)
    return scale, beta - mean * scale


def _band_cmajor(w_dhwio, scale, wo, pad):
    k = w_dhwio.shape[0]
    ci, co = w_dhwio.shape[3], w_dhwio.shape[4]
    t = k * k
    L = wo + k
    wt = jnp.transpose(w_dhwio * scale, (0, 1, 3, 2, 4))
    wt = wt.reshape(t, ci, k, co).astype(jnp.bfloat16)
    r = jnp.flip(wt, axis=2)
    r = jnp.pad(r, ((0, 0), (0, 0), (0, L + 1 - k), (0, 0)))
    g = jnp.broadcast_to(r[:, :, None], (t, ci, wo, L + 1, co))
    g = g.reshape(t, ci, wo * (L + 1) * co)[:, :, : wo * L * co]
    g = g.reshape(t, ci, wo, L, co)[:, :, :, k - 1 - pad: k - 1 - pad + wo, :]
    return g.reshape(t, ci * wo, wo * co)


def _band_wmajor(w_dhwio, scale, wo, pad):
    k = w_dhwio.shape[0]
    ci, co = w_dhwio.shape[3], w_dhwio.shape[4]
    t = k * k
    L = wo + k
    wt = jnp.transpose(w_dhwio * scale, (0, 1, 4, 2, 3))
    wt = wt.reshape(t, co, k, ci).astype(jnp.bfloat16)
    zmid = jnp.zeros((t, co, L + 1 - k, ci), wt.dtype)
    r = jnp.concatenate([wt[:, :, pad:, :], zmid, wt[:, :, :pad, :]], axis=2)
    g = jnp.broadcast_to(r[:, :, None], (t, co, wo, L + 1, ci))
    g = g.reshape(t, co, wo * (L + 1) * ci)[:, :, : wo * L * ci]
    g = g.reshape(t, co, wo, L, ci)[:, :, :, :wo, :]
    g = jnp.transpose(g, (0, 3, 4, 2, 1))
    return g.reshape(t, wo * ci, wo * co)


def _fused_block_kernel(xp_ref, w1_ref, wsc_ref, w2_ref, b1_ref, b2_ref,
                        b3_ref, y_ref, h_scr, sc_scr, *, nb, do, ho, kin,
                        lanes):
    rows = do * ho

    for b in range(nb):
        acc1 = jnp.zeros((rows, lanes), jnp.float32)
        accs = jnp.zeros((rows, lanes), jnp.float32)
        for kh in range(5):
            xh = xp_ref[b, :, pl.ds(kh, ho), :]
            for kd in range(5):
                s = xh[kd:kd + do].reshape(rows, kin)
                accs = accs + jnp.dot(s, wsc_ref[kd * 5 + kh],
                                      preferred_element_type=jnp.float32)
                if 1 <= kd <= 3 and 1 <= kh <= 3:
                    acc1 = acc1 + jnp.dot(s, w1_ref[(kd - 1) * 3 + (kh - 1)],
                                          preferred_element_type=jnp.float32)
        sc_scr[b] = accs + b3_ref[...]
        h = jnp.maximum(acc1 + b1_ref[...], 0.0).astype(jnp.bfloat16)

        h_scr[b] = jnp.zeros((do + 2, ho + 2, lanes), jnp.bfloat16)
        h_scr[b, pl.ds(1, do), pl.ds(1, ho), :] = h.reshape(do, ho, lanes)

    for b in range(nb):
        acc2 = jnp.zeros((rows, lanes), jnp.float32)
        for kh in range(3):
            hh = h_scr[b, :, pl.ds(kh, ho), :]
            for kd in range(3):
                s = hh[kd:kd + do].reshape(rows, lanes)
                acc2 = acc2 + jnp.dot(s, w2_ref[kd * 3 + kh],
                                      preferred_element_type=jnp.float32)
        y_ref[b] = jnp.maximum(acc2 + b2_ref[...] + sc_scr[b], 0.0)


def kernel(x, w1, bn1_gamma, bn1_beta, bn1_mean, bn1_var,
           w2, bn2_gamma, bn2_beta, bn2_mean, bn2_var,
           w_sc, bn3_gamma, bn3_beta, bn3_mean, bn3_var):
    B, Cin, D, H, W = x.shape
    Cout = w1.shape[-1]
    Do, Ho, Wo = D, H, W
    kin = W * Cin
    lanes = Wo * Cout
    rows = Do * Ho

    x_cl = jnp.transpose(x.astype(jnp.bfloat16), (0, 2, 3, 1, 4))
    x_cl = x_cl.reshape(B, D, H, kin)
    xp = jnp.pad(x_cl, ((0, 0), (2, 2), (2, 2), (0, 0)))
    Dp, Hp = D + 4, H + 4

    s1, c1 = _fold_bn(bn1_gamma, bn1_beta, bn1_mean, bn1_var)
    s2, c2 = _fold_bn(bn2_gamma, bn2_beta, bn2_mean, bn2_var)
    s3, c3 = _fold_bn(bn3_gamma, bn3_beta, bn3_mean, bn3_var)

    w1b = _band_cmajor(w1, s1, Wo, 1)
    wscb = _band_cmajor(w_sc, s3, Wo, 2)
    w2b = _band_wmajor(w2, s2, Wo, 1)
    b1t = jnp.tile(c1, Wo).reshape(1, lanes).astype(jnp.float32)
    b2t = jnp.tile(c2, Wo).reshape(1, lanes).astype(jnp.float32)
    b3t = jnp.tile(c3, Wo).reshape(1, lanes).astype(jnp.float32)

    nb = 2 if B % 2 == 0 else 1
    kern = partial(_fused_block_kernel, nb=nb, do=Do, ho=Ho, kin=kin,
                   lanes=lanes)
    flops = 2 * B * rows * (kin * 34 + lanes * 9) * lanes
    bytes_accessed = int(xp.size * 2 + (w1b.size + wscb.size + w2b.size) * 2
                         + B * rows * lanes * 4)

    y = pl.pallas_call(
        kern,
        out_shape=jax.ShapeDtypeStruct((B, rows, lanes), jnp.float32),
        grid=(B // nb,),
        in_specs=[
            pl.BlockSpec((nb, Dp, Hp, kin), lambda b: (b, 0, 0, 0)),
            pl.BlockSpec((9, kin, lanes), lambda b: (0, 0, 0)),
            pl.BlockSpec((25, kin, lanes), lambda b: (0, 0, 0)),
            pl.BlockSpec((9, lanes, lanes), lambda b: (0, 0, 0)),
            pl.BlockSpec((1, lanes), lambda b: (0, 0)),
            pl.BlockSpec((1, lanes), lambda b: (0, 0)),
            pl.BlockSpec((1, lanes), lambda b: (0, 0)),
        ],
        out_specs=pl.BlockSpec((nb, rows, lanes), lambda b: (b, 0, 0)),
        scratch_shapes=[
            pltpu.VMEM((nb, Do + 2, Ho + 2, lanes), jnp.bfloat16),
            pltpu.VMEM((nb, rows, lanes), jnp.float32),
        ],
        compiler_params=pltpu.CompilerParams(
            dimension_semantics=("parallel",),
            vmem_limit_bytes=64 * 1024 * 1024,
        ),
        cost_estimate=pl.CostEstimate(flops=flops, transcendentals=0,
                                      bytes_accessed=bytes_accessed),
    )(xp, w1b, wscb, w2b, b1t, b2t, b3t)

    y = y.reshape(B, Do, Ho, Wo, Cout)
    return jnp.transpose(y, (0, 4, 1, 2, 3))


# speedup vs baseline: 1.5311x; 1.5311x over previous
from functools import partial

import jax
import jax.numpy as jnp
from jax.experimental import pallas as pl
from jax.experimental.pallas import tpu as pltpu


def _fold_bn(gamma, beta, mean, var, eps=1e-5):
    scale = gamma / jnp.sqrt(var + eps)
    return scale, beta - mean * scale


def _band_builder_kernel(wt_ref, out_ref, *, k, pad, wo, ci, co, cmajor):
    n_r, n_c = wo * ci, wo * co
    ri = jax.lax.broadcasted_iota(jnp.int32, (n_r, n_c), 0)
    cj = jax.lax.broadcasted_iota(jnp.int32, (n_r, n_c), 1)
    w_in = ri % wo if cmajor else ri // ci
    kw_map = w_in - (cj // co) + pad
    acc = jnp.zeros((n_r, n_c), jnp.bfloat16)
    for kw in range(k):
        wk = wt_ref[kw]
        if cmajor:
            tfull = jnp.broadcast_to(wk[:, None, :], (ci, wo, n_c))
        else:
            tfull = jnp.broadcast_to(wk[None, :, :], (wo, ci, n_c))
        acc = jnp.where(kw_map == kw, tfull.reshape(n_r, n_c), acc)
    out_ref[...] = acc


def _build_band(w_dhwio, scale, wo, pad, cmajor):
    k = w_dhwio.shape[0]
    ci, co = w_dhwio.shape[3], w_dhwio.shape[4]
    t = k * k
    wt = (w_dhwio * scale).reshape(t, k, ci, co).astype(jnp.bfloat16)
    wt = jnp.tile(wt, (1, 1, 1, wo))
    return pl.pallas_call(
        partial(_band_builder_kernel, k=k, pad=pad, wo=wo, ci=ci, co=co,
                cmajor=cmajor),
        out_shape=jax.ShapeDtypeStruct((t, wo * ci, wo * co), jnp.bfloat16),
        grid=(t,),
        in_specs=[pl.BlockSpec((None, k, ci, wo * co),
                               lambda i: (i, 0, 0, 0))],
        out_specs=pl.BlockSpec((None, wo * ci, wo * co), lambda i: (i, 0, 0)),
        compiler_params=pltpu.CompilerParams(
            dimension_semantics=("parallel",),
        ),
    )(wt)


def _fused_block_kernel(xp_ref, w1_ref, wsc_ref, w2_ref, b1_ref, b2_ref,
                        b3_ref, y_ref, h_scr, sc_scr, *, nb, do, ho, kin,
                        lanes):
    rows = do * ho

    for b in range(nb):
        acc1 = jnp.zeros((rows, lanes), jnp.float32)
        accs = jnp.zeros((rows, lanes), jnp.float32)
        for kh in range(5):
            xh = xp_ref[b, :, pl.ds(kh, ho), :]
            for kd in range(5):
                s = xh[kd:kd + do].reshape(rows, kin)
                accs = accs + jnp.dot(s, wsc_ref[kd * 5 + kh],
                                      preferred_element_type=jnp.float32)
                if 1 <= kd <= 3 and 1 <= kh <= 3:
                    acc1 = acc1 + jnp.dot(s, w1_ref[(kd - 1) * 3 + (kh - 1)],
                                          preferred_element_type=jnp.float32)
        sc_scr[b] = accs + b3_ref[...]
        h = jnp.maximum(acc1 + b1_ref[...], 0.0).astype(jnp.bfloat16)

        h_scr[b] = jnp.zeros((do + 2, ho + 2, lanes), jnp.bfloat16)
        h_scr[b, pl.ds(1, do), pl.ds(1, ho), :] = h.reshape(do, ho, lanes)

    for b in range(nb):
        acc2 = jnp.zeros((rows, lanes), jnp.float32)
        for kh in range(3):
            hh = h_scr[b, :, pl.ds(kh, ho), :]
            for kd in range(3):
                s = hh[kd:kd + do].reshape(rows, lanes)
                acc2 = acc2 + jnp.dot(s, w2_ref[kd * 3 + kh],
                                      preferred_element_type=jnp.float32)
        y_ref[b] = jnp.maximum(acc2 + b2_ref[...] + sc_scr[b], 0.0)


def kernel(x, w1, bn1_gamma, bn1_beta, bn1_mean, bn1_var,
           w2, bn2_gamma, bn2_beta, bn2_mean, bn2_var,
           w_sc, bn3_gamma, bn3_beta, bn3_mean, bn3_var):
    B, Cin, D, H, W = x.shape
    Cout = w1.shape[-1]
    Do, Ho, Wo = D, H, W
    kin = W * Cin
    lanes = Wo * Cout
    rows = Do * Ho

    x_cl = jnp.transpose(x.astype(jnp.bfloat16), (0, 2, 3, 1, 4))
    x_cl = x_cl.reshape(B, D, H, kin)
    xp = jnp.pad(x_cl, ((0, 0), (2, 2), (2, 2), (0, 0)))
    Dp, Hp = D + 4, H + 4

    s1, c1 = _fold_bn(bn1_gamma, bn1_beta, bn1_mean, bn1_var)
    s2, c2 = _fold_bn(bn2_gamma, bn2_beta, bn2_mean, bn2_var)
    s3, c3 = _fold_bn(bn3_gamma, bn3_beta, bn3_mean, bn3_var)

    w1b = _build_band(w1, s1, Wo, 1, True)
    wscb = _build_band(w_sc, s3, Wo, 2, True)
    w2b = _build_band(w2, s2, Wo, 1, False)
    b1t = jnp.tile(c1, Wo).reshape(1, lanes).astype(jnp.float32)
    b2t = jnp.tile(c2, Wo).reshape(1, lanes).astype(jnp.float32)
    b3t = jnp.tile(c3, Wo).reshape(1, lanes).astype(jnp.float32)

    nb = 4 if B % 4 == 0 else (2 if B % 2 == 0 else 1)
    kern = partial(_fused_block_kernel, nb=nb, do=Do, ho=Ho, kin=kin,
                   lanes=lanes)
    flops = 2 * B * rows * (kin * 34 + lanes * 9) * lanes
    bytes_accessed = int(xp.size * 2 + (w1b.size + wscb.size + w2b.size) * 2
                         + B * rows * lanes * 4)

    y = pl.pallas_call(
        kern,
        out_shape=jax.ShapeDtypeStruct((B, rows, lanes), jnp.float32),
        grid=(B // nb,),
        in_specs=[
            pl.BlockSpec((nb, Dp, Hp, kin), lambda b: (b, 0, 0, 0)),
            pl.BlockSpec((9, kin, lanes), lambda b: (0, 0, 0)),
            pl.BlockSpec((25, kin, lanes), lambda b: (0, 0, 0)),
            pl.BlockSpec((9, lanes, lanes), lambda b: (0, 0, 0)),
            pl.BlockSpec((1, lanes), lambda b: (0, 0)),
            pl.BlockSpec((1, lanes), lambda b: (0, 0)),
            pl.BlockSpec((1, lanes), lambda b: (0, 0)),
        ],
        out_specs=pl.BlockSpec((nb, rows, lanes), lambda b: (b, 0, 0)),
        scratch_shapes=[
            pltpu.VMEM((nb, Do + 2, Ho + 2, lanes), jnp.bfloat16),
            pltpu.VMEM((nb, rows, lanes), jnp.float32),
        ],
        compiler_params=pltpu.CompilerParams(
            dimension_semantics=("parallel",),
            vmem_limit_bytes=64 * 1024 * 1024,
        ),
        cost_estimate=pl.CostEstimate(flops=flops, transcendentals=0,
                                      bytes_accessed=bytes_accessed),
    )(xp, w1b, wscb, w2b, b1t, b2t, b3t)

    y = y.reshape(B, Do, Ho, Wo, Cout)
    return jnp.transpose(y, (0, 4, 1, 2, 3))


# speedup vs baseline: 1.5507x; 1.0128x over previous
from functools import partial

import jax
import jax.numpy as jnp
from jax.experimental import pallas as pl
from jax.experimental.pallas import tpu as pltpu


def _fold_bn(gamma, beta, mean, var, eps=1e-5):
    scale = gamma / jnp.sqrt(var + eps)
    return scale, beta - mean * scale


def _band_builder_kernel(wt_ref, out_ref, *, k, pad, wo, ci, co, cmajor):
    n_r, n_c = wo * ci, wo * co
    ri = jax.lax.broadcasted_iota(jnp.int32, (n_r, n_c), 0)
    cj = jax.lax.broadcasted_iota(jnp.int32, (n_r, n_c), 1)
    w_in = ri % wo if cmajor else ri // ci
    kw_map = w_in - (cj // co) + pad
    acc = jnp.zeros((n_r, n_c), jnp.bfloat16)
    for kw in range(k):
        wk = wt_ref[kw]
        if cmajor:
            tfull = jnp.broadcast_to(wk[:, None, :], (ci, wo, n_c))
        else:
            tfull = jnp.broadcast_to(wk[None, :, :], (wo, ci, n_c))
        acc = jnp.where(kw_map == kw, tfull.reshape(n_r, n_c), acc)
    out_ref[...] = acc


def _build_band(w_dhwio, scale, wo, pad, cmajor):
    k = w_dhwio.shape[0]
    ci, co = w_dhwio.shape[3], w_dhwio.shape[4]
    t = k * k
    wt = (w_dhwio * scale).reshape(t, k, ci, co).astype(jnp.bfloat16)
    wt = jnp.tile(wt, (1, 1, 1, wo))
    return pl.pallas_call(
        partial(_band_builder_kernel, k=k, pad=pad, wo=wo, ci=ci, co=co,
                cmajor=cmajor),
        out_shape=jax.ShapeDtypeStruct((t, wo * ci, wo * co), jnp.bfloat16),
        grid=(t,),
        in_specs=[pl.BlockSpec((None, k, ci, wo * co),
                               lambda i: (i, 0, 0, 0))],
        out_specs=pl.BlockSpec((None, wo * ci, wo * co), lambda i: (i, 0, 0)),
        compiler_params=pltpu.CompilerParams(
            dimension_semantics=("parallel",),
        ),
    )(wt)


def _fused_block_kernel(xp_ref, w1_ref, wsc_ref, w2_ref, b1_ref, b2_ref,
                        b3_ref, y_ref, h_scr, sc_scr, *, nb, do, ho, kin,
                        lanes):
    rows = do * ho

    for b in range(nb):
        acc1 = jnp.zeros((rows, lanes), jnp.float32)
        accs = jnp.zeros((rows, lanes), jnp.float32)
        for kh in range(5):
            xh = xp_ref[b, :, pl.ds(kh, ho), :]
            for kd in range(5):
                s = xh[kd:kd + do].reshape(rows, kin)
                accs = accs + jnp.dot(s, wsc_ref[kd * 5 + kh],
                                      preferred_element_type=jnp.float32)
                if 1 <= kd <= 3 and 1 <= kh <= 3:
                    acc1 = acc1 + jnp.dot(s, w1_ref[(kd - 1) * 3 + (kh - 1)],
                                          preferred_element_type=jnp.float32)
        sc_scr[b] = accs + b3_ref[...]
        h = jnp.maximum(acc1 + b1_ref[...], 0.0).astype(jnp.bfloat16)

        h_scr[b] = jnp.zeros((do + 2, ho + 2, lanes), jnp.bfloat16)
        h_scr[b, pl.ds(1, do), pl.ds(1, ho), :] = h.reshape(do, ho, lanes)

    for b in range(nb):
        acc2 = jnp.zeros((rows, lanes), jnp.float32)
        for kh in range(3):
            hh = h_scr[b, :, pl.ds(kh, ho), :]
            for kd in range(3):
                s = hh[kd:kd + do].reshape(rows, lanes)
                acc2 = acc2 + jnp.dot(s, w2_ref[kd * 3 + kh],
                                      preferred_element_type=jnp.float32)
        y_ref[b] = jnp.maximum(acc2 + b2_ref[...] + sc_scr[b],
                               0.0).astype(jnp.bfloat16)


def kernel(x, w1, bn1_gamma, bn1_beta, bn1_mean, bn1_var,
           w2, bn2_gamma, bn2_beta, bn2_mean, bn2_var,
           w_sc, bn3_gamma, bn3_beta, bn3_mean, bn3_var):
    B, Cin, D, H, W = x.shape
    Cout = w1.shape[-1]
    Do, Ho, Wo = D, H, W
    kin = W * Cin
    lanes = Wo * Cout
    rows = Do * Ho

    x_cl = jnp.transpose(x.astype(jnp.bfloat16), (0, 2, 3, 1, 4))
    x_cl = x_cl.reshape(B, D, H, kin)
    xp = jnp.pad(x_cl, ((0, 0), (2, 2), (2, 2), (0, 0)))
    Dp, Hp = D + 4, H + 4

    s1, c1 = _fold_bn(bn1_gamma, bn1_beta, bn1_mean, bn1_var)
    s2, c2 = _fold_bn(bn2_gamma, bn2_beta, bn2_mean, bn2_var)
    s3, c3 = _fold_bn(bn3_gamma, bn3_beta, bn3_mean, bn3_var)

    w1b = _build_band(w1, s1, Wo, 1, True)
    wscb = _build_band(w_sc, s3, Wo, 2, True)
    w2b = _build_band(w2, s2, Wo, 1, False)
    b1t = jnp.tile(c1, Wo).reshape(1, lanes).astype(jnp.float32)
    b2t = jnp.tile(c2, Wo).reshape(1, lanes).astype(jnp.float32)
    b3t = jnp.tile(c3, Wo).reshape(1, lanes).astype(jnp.float32)

    nb = 4 if B % 4 == 0 else (2 if B % 2 == 0 else 1)
    kern = partial(_fused_block_kernel, nb=nb, do=Do, ho=Ho, kin=kin,
                   lanes=lanes)
    flops = 2 * B * rows * (kin * 34 + lanes * 9) * lanes
    bytes_accessed = int(xp.size * 2 + (w1b.size + wscb.size + w2b.size) * 2
                         + B * rows * lanes * 4)

    y = pl.pallas_call(
        kern,
        out_shape=jax.ShapeDtypeStruct((B, rows, lanes), jnp.bfloat16),
        grid=(B // nb,),
        in_specs=[
            pl.BlockSpec((nb, Dp, Hp, kin), lambda b: (b, 0, 0, 0)),
            pl.BlockSpec((9, kin, lanes), lambda b: (0, 0, 0)),
            pl.BlockSpec((25, kin, lanes), lambda b: (0, 0, 0)),
            pl.BlockSpec((9, lanes, lanes), lambda b: (0, 0, 0)),
            pl.BlockSpec((1, lanes), lambda b: (0, 0)),
            pl.BlockSpec((1, lanes), lambda b: (0, 0)),
            pl.BlockSpec((1, lanes), lambda b: (0, 0)),
        ],
        out_specs=pl.BlockSpec((nb, rows, lanes), lambda b: (b, 0, 0)),
        scratch_shapes=[
            pltpu.VMEM((nb, Do + 2, Ho + 2, lanes), jnp.bfloat16),
            pltpu.VMEM((nb, rows, lanes), jnp.float32),
        ],
        compiler_params=pltpu.CompilerParams(
            dimension_semantics=("parallel",),
            vmem_limit_bytes=64 * 1024 * 1024,
        ),
        cost_estimate=pl.CostEstimate(flops=flops, transcendentals=0,
                                      bytes_accessed=bytes_accessed),
    )(xp, w1b, wscb, w2b, b1t, b2t, b3t)

    y = y.reshape(B, Do, Ho, Wo, Cout)
    return jnp.transpose(y, (0, 4, 1, 2, 3)).astype(jnp.float32)


# speedup vs baseline: 1.5832x; 1.0210x over previous
from functools import partial

import jax
import jax.numpy as jnp
from jax.experimental import pallas as pl
from jax.experimental.pallas import tpu as pltpu


def _fold_bn(gamma, beta, mean, var, eps=1e-5):
    scale = gamma / jnp.sqrt(var + eps)
    return scale, beta - mean * scale


def _band_builder_kernel(wt_ref, out_ref, *, k, pad, wo, ci, co, cmajor):
    n_r, n_c = wo * ci, wo * co
    ri = jax.lax.broadcasted_iota(jnp.int32, (n_r, n_c), 0)
    cj = jax.lax.broadcasted_iota(jnp.int32, (n_r, n_c), 1)
    w_in = ri % wo if cmajor else ri // ci
    kw_map = w_in - (cj // co) + pad
    acc = jnp.zeros((n_r, n_c), jnp.bfloat16)
    for kw in range(k):
        wk = wt_ref[kw]
        if cmajor:
            tfull = jnp.broadcast_to(wk[:, None, :], (ci, wo, n_c))
        else:
            tfull = jnp.broadcast_to(wk[None, :, :], (wo, ci, n_c))
        acc = jnp.where(kw_map == kw, tfull.reshape(n_r, n_c), acc)
    out_ref[...] = acc


def _build_band(w_dhwio, scale, wo, pad, cmajor):
    k = w_dhwio.shape[0]
    ci, co = w_dhwio.shape[3], w_dhwio.shape[4]
    t = k * k
    wt = (w_dhwio * scale).reshape(t, k, ci, co).astype(jnp.bfloat16)
    wt = jnp.tile(wt, (1, 1, 1, wo))
    return pl.pallas_call(
        partial(_band_builder_kernel, k=k, pad=pad, wo=wo, ci=ci, co=co,
                cmajor=cmajor),
        out_shape=jax.ShapeDtypeStruct((t, wo * ci, wo * co), jnp.bfloat16),
        grid=(t,),
        in_specs=[pl.BlockSpec((None, k, ci, wo * co),
                               lambda i: (i, 0, 0, 0))],
        out_specs=pl.BlockSpec((None, wo * ci, wo * co), lambda i: (i, 0, 0)),
        compiler_params=pltpu.CompilerParams(
            dimension_semantics=("parallel",),
        ),
    )(wt)


def _fused_block_kernel(xp_ref, w1_ref, wsc_ref, w2_ref, b1_ref, b2_ref,
                        b3_ref, y_ref, h_scr, sc_scr, *, nb, do, ho, kin,
                        lanes):
    rows = do * ho

    for b in range(nb):
        acc1 = jnp.zeros((rows, lanes), jnp.float32)
        accs = jnp.zeros((rows, lanes), jnp.float32)
        for kh in range(5):
            xh = xp_ref[b, :, pl.ds(kh, ho), :]
            for kd in range(5):
                s = xh[kd:kd + do].reshape(rows, kin)
                accs = accs + jnp.dot(s, wsc_ref[kd * 5 + kh],
                                      preferred_element_type=jnp.float32)
                if 1 <= kd <= 3 and 1 <= kh <= 3:
                    acc1 = acc1 + jnp.dot(s, w1_ref[(kd - 1) * 3 + (kh - 1)],
                                          preferred_element_type=jnp.float32)
        sc_scr[b] = accs + b3_ref[...]
        h = jnp.maximum(acc1 + b1_ref[...], 0.0).astype(jnp.bfloat16)

        h_scr[b] = jnp.zeros((do + 2, ho + 2, lanes), jnp.bfloat16)
        h_scr[b, pl.ds(1, do), pl.ds(1, ho), :] = h.reshape(do, ho, lanes)

    for b in range(nb):
        acc2 = jnp.zeros((rows, lanes), jnp.float32)
        for kh in range(3):
            hh = h_scr[b, :, pl.ds(kh, ho), :]
            for kd in range(3):
                s = hh[kd:kd + do].reshape(rows, lanes)
                acc2 = acc2 + jnp.dot(s, w2_ref[kd * 3 + kh],
                                      preferred_element_type=jnp.float32)
        y_ref[b] = jnp.maximum(acc2 + b2_ref[...] + sc_scr[b],
                               0.0).astype(jnp.bfloat16)


def kernel(x, w1, bn1_gamma, bn1_beta, bn1_mean, bn1_var,
           w2, bn2_gamma, bn2_beta, bn2_mean, bn2_var,
           w_sc, bn3_gamma, bn3_beta, bn3_mean, bn3_var):
    B, Cin, D, H, W = x.shape
    Cout = w1.shape[-1]
    Do, Ho, Wo = D, H, W
    kin = W * Cin
    lanes = Wo * Cout
    rows = Do * Ho

    x_cl = jnp.transpose(x.astype(jnp.bfloat16), (0, 2, 3, 1, 4))
    x_cl = x_cl.reshape(B, D, H, kin)
    xp = jnp.pad(x_cl, ((0, 0), (2, 2), (2, 2), (0, 0)))
    Dp, Hp = D + 4, H + 4

    s1, c1 = _fold_bn(bn1_gamma, bn1_beta, bn1_mean, bn1_var)
    s2, c2 = _fold_bn(bn2_gamma, bn2_beta, bn2_mean, bn2_var)
    s3, c3 = _fold_bn(bn3_gamma, bn3_beta, bn3_mean, bn3_var)

    w1b = _build_band(w1, s1, Wo, 1, True)
    wscb = _build_band(w_sc, s3, Wo, 2, True)
    w2b = _build_band(w2, s2, Wo, 1, False)
    b1t = jnp.tile(c1, Wo).reshape(1, lanes).astype(jnp.float32)
    b2t = jnp.tile(c2, Wo).reshape(1, lanes).astype(jnp.float32)
    b3t = jnp.tile(c3, Wo).reshape(1, lanes).astype(jnp.float32)

    nb = 8 if B % 8 == 0 else (2 if B % 2 == 0 else 1)
    kern = partial(_fused_block_kernel, nb=nb, do=Do, ho=Ho, kin=kin,
                   lanes=lanes)
    flops = 2 * B * rows * (kin * 34 + lanes * 9) * lanes
    bytes_accessed = int(xp.size * 2 + (w1b.size + wscb.size + w2b.size) * 2
                         + B * rows * lanes * 4)

    y = pl.pallas_call(
        kern,
        out_shape=jax.ShapeDtypeStruct((B, rows, lanes), jnp.bfloat16),
        grid=(B // nb,),
        in_specs=[
            pl.BlockSpec((nb, Dp, Hp, kin), lambda b: (b, 0, 0, 0)),
            pl.BlockSpec((9, kin, lanes), lambda b: (0, 0, 0)),
            pl.BlockSpec((25, kin, lanes), lambda b: (0, 0, 0)),
            pl.BlockSpec((9, lanes, lanes), lambda b: (0, 0, 0)),
            pl.BlockSpec((1, lanes), lambda b: (0, 0)),
            pl.BlockSpec((1, lanes), lambda b: (0, 0)),
            pl.BlockSpec((1, lanes), lambda b: (0, 0)),
        ],
        out_specs=pl.BlockSpec((nb, rows, lanes), lambda b: (b, 0, 0)),
        scratch_shapes=[
            pltpu.VMEM((nb, Do + 2, Ho + 2, lanes), jnp.bfloat16),
            pltpu.VMEM((nb, rows, lanes), jnp.float32),
        ],
        compiler_params=pltpu.CompilerParams(
            dimension_semantics=("parallel",),
            vmem_limit_bytes=64 * 1024 * 1024,
        ),
        cost_estimate=pl.CostEstimate(flops=flops, transcendentals=0,
                                      bytes_accessed=bytes_accessed),
    )(xp, w1b, wscb, w2b, b1t, b2t, b3t)

    y = y.reshape(B, Do, Ho, Wo, Cout)
    return jnp.transpose(y, (0, 4, 1, 2, 3)).astype(jnp.float32)
